# Initial kernel scaffold; baseline (speedup 1.0000x reference)
import jax
import jax.numpy as jnp
from jax.experimental import pallas as pl


def kernel(tokens, start_pos, tok_embeddings_weight):
    raise NotImplementedError("write your pallas kernel here")


# speedup vs baseline: 1.6246x; 1.6246x over previous
import functools

import jax
import jax.numpy as jnp
from jax import lax
from jax.experimental import pallas as pl
from jax.experimental.pallas import tpu as pltpu
from jax.experimental.pallas import tpu_sc as plsc


def _emb_lookup(n_tokens, vocab, dim, num_workers, chunk):
    n_per_w = n_tokens // num_workers
    n_chunks = n_per_w // chunk
    mesh = plsc.VectorSubcoreMesh(core_axis_name="c", subcore_axis_name="s")
    num_cores = 2

    @functools.partial(
        pl.kernel,
        mesh=mesh,
        out_type=jax.ShapeDtypeStruct((n_tokens, dim), jnp.float32),
        scratch_types=[
            pltpu.VMEM((n_per_w,), jnp.int32),
            pltpu.VMEM((chunk, dim), jnp.float32),
            pltpu.SemaphoreType.DMA,
        ],
    )
    def emb(tok_hbm, table_hbm, out_hbm, idx_v, rows_v, gsem):
        wid = lax.axis_index("s") * num_cores + lax.axis_index("c")
        base = wid * n_per_w
        pltpu.sync_copy(tok_hbm.at[pl.ds(base, n_per_w)], idx_v)

        def body(g, carry):
            off = g * chunk
            pltpu.async_copy(
                table_hbm.at[idx_v.at[pl.ds(off, chunk)]], rows_v, gsem
            ).wait()
            pltpu.sync_copy(rows_v, out_hbm.at[pl.ds(base + off, chunk)])
            return carry

        lax.fori_loop(0, n_chunks, body, 0)

    return emb


def kernel(tokens, start_pos, tok_embeddings_weight):
    b, s = tokens.shape
    v, d = tok_embeddings_weight.shape
    n = b * s
    flat = tokens.reshape(n)
    emb = _emb_lookup(n, v, d, num_workers=32, chunk=64)
    out = emb(flat, tok_embeddings_weight)
    return out.reshape(b, s, d)


# speedup vs baseline: 1.6782x; 1.0330x over previous
import functools

import jax
import jax.numpy as jnp
from jax import lax
from jax.experimental import pallas as pl
from jax.experimental.pallas import tpu as pltpu
from jax.experimental.pallas import tpu_sc as plsc


def _emb_lookup(n_tokens, vocab, dim, num_workers, chunk):
    n_per_w = n_tokens // num_workers
    n_chunks = n_per_w // chunk
    assert n_chunks % 2 == 0 and n_chunks >= 4
    mesh = plsc.VectorSubcoreMesh(core_axis_name="c", subcore_axis_name="s")
    num_cores = 2

    @functools.partial(
        pl.kernel,
        mesh=mesh,
        out_type=jax.ShapeDtypeStruct((n_tokens, dim), jnp.float32),
        scratch_types=[
            pltpu.VMEM((n_per_w,), jnp.int32),
            pltpu.VMEM((chunk, dim), jnp.float32),
            pltpu.VMEM((chunk, dim), jnp.float32),
            pltpu.SemaphoreType.DMA,
            pltpu.SemaphoreType.DMA,
            pltpu.SemaphoreType.DMA,
            pltpu.SemaphoreType.DMA,
        ],
    )
    def emb(tok_hbm, table_hbm, out_hbm, idx_v, rows0, rows1, g0, g1, w0, w1):
        wid = lax.axis_index("s") * num_cores + lax.axis_index("c")
        base = wid * n_per_w
        pltpu.sync_copy(tok_hbm.at[pl.ds(base, n_per_w)], idx_v)

        rows = (rows0, rows1)
        gsem = (g0, g1)
        wsem = (w0, w1)

        def gather_desc(g, b):
            return pltpu.make_async_copy(
                table_hbm.at[idx_v.at[pl.ds(g * chunk, chunk)]], rows[b], gsem[b]
            )

        def write_desc(g, b):
            return pltpu.make_async_copy(
                rows[b], out_hbm.at[pl.ds(base + g * chunk, chunk)], wsem[b]
            )

        gather_desc(0, 0).start()
        gather_desc(1, 1).start()

        def body(i, carry):
            g2 = i * 2
            for b in range(2):
                gather_desc(g2 + b, b).wait()
                write_desc(g2 + b, b).start()
            for b in range(2):
                write_desc(g2 + b, b).wait()
                gather_desc(g2 + 2 + b, b).start()
            return carry

        lax.fori_loop(0, n_chunks // 2 - 1, body, 0)

        last = n_chunks - 2
        for b in range(2):
            gather_desc(last + b, b).wait()
            write_desc(last + b, b).start()
        for b in range(2):
            write_desc(last + b, b).wait()

    return emb


def kernel(tokens, start_pos, tok_embeddings_weight):
    b, s = tokens.shape
    v, d = tok_embeddings_weight.shape
    n = b * s
    flat = tokens.reshape(n)
    emb = _emb_lookup(n, v, d, num_workers=32, chunk=32)
    out = emb(flat, tok_embeddings_weight)
    return out.reshape(b, s, d)
